# Initial kernel scaffold; baseline (speedup 1.0000x reference)
import jax
import jax.numpy as jnp
from jax.experimental import pallas as pl


def kernel(x, edge_index, Wz, bz, Lz_w, Lz_b, Wr, br, Lr_w, Lr_b, Wh, bh, Lh_w, Lh_b, Wout, bout):
    raise NotImplementedError("write your pallas kernel here")


# speedup vs baseline: 40.6686x; 40.6686x over previous
import jax
import jax.numpy as jnp
from jax import lax
from jax.experimental import pallas as pl
from jax.experimental.pallas import tpu as pltpu
from jax.experimental.pallas import tpu_sc as plsc

N = 10000
C = 128
E = 320000
OUT_DIM = 45

NC = 2
NS = 16
NW = NC * NS
NPAD = 10240
STRIPE = NPAD // NS
EPW = E // NW
CH = 80
K = EPW // CH

_MESH = plsc.VectorSubcoreMesh(core_axis_name="c", subcore_axis_name="s")


def _deg_body(dst_hbm, zer_hbm, one_hbm, out_hbm, deg_sp, idx_v, one_v):
    c = lax.axis_index("c")
    s = lax.axis_index("s")
    w = c * NS + s
    pltpu.sync_copy(zer_hbm, deg_sp.at[pl.ds(s * STRIPE, STRIPE)])
    pltpu.sync_copy(one_hbm, one_v)
    pltpu.sync_copy(dst_hbm.at[w], idx_v)
    plsc.subcore_barrier()

    def body(k, carry):
        pltpu.sync_copy(one_v, deg_sp.at[idx_v.at[k]], add=True)
        return carry

    lax.fori_loop(0, K, body, 0)
    plsc.subcore_barrier()
    pltpu.sync_copy(deg_sp.at[pl.ds(s * STRIPE, STRIPE)],
                    out_hbm.at[c, pl.ds(s * STRIPE, STRIPE)])


def _scatter_body(src_hbm, dst_hbm, xs_hbm, zrow_hbm, out_hbm,
                  acc_sp, src_v, dst_v, rows_v, sem):
    c = lax.axis_index("c")
    s = lax.axis_index("s")
    w = c * NS + s
    pltpu.sync_copy(zrow_hbm, acc_sp.at[pl.ds(s * STRIPE, STRIPE), :])
    pltpu.sync_copy(src_hbm.at[w], src_v)
    pltpu.sync_copy(dst_hbm.at[w], dst_v)
    plsc.subcore_barrier()

    def body(k, carry):
        pltpu.async_copy(xs_hbm.at[src_v.at[k]], rows_v, sem).wait()
        pltpu.sync_copy(rows_v, acc_sp.at[dst_v.at[k]], add=True)
        return carry

    lax.fori_loop(0, K, body, 0)
    plsc.subcore_barrier()
    pltpu.sync_copy(acc_sp.at[pl.ds(s * STRIPE, STRIPE), :],
                    out_hbm.at[c, pl.ds(s * STRIPE, STRIPE), :])


def _scale_body(degp, xr, wz, lz1, wh, lh1, bz2, lzb2, bh2, lhb2,
                xs, wcat, cc):
    d = degp[0] + degp[1] + 1.0
    dinv = lax.rsqrt(d)
    xs[...] = xr[...] * dinv

    @pl.when(pl.program_id(0) == 0)
    def _():
        az = jnp.dot(wz[...], lz1[...], preferred_element_type=jnp.float32)
        ah = jnp.dot(wh[...], lh1[...], preferred_element_type=jnp.float32)
        wcat[...] = jnp.concatenate([az, ah], axis=1)
        cz = jnp.dot(bz2[...], lz1[...], preferred_element_type=jnp.float32) + lzb2[...]
        chh = jnp.dot(bh2[...], lh1[...], preferred_element_type=jnp.float32) + lhb2[...]
        cc[...] = jnp.concatenate([cz, chh], axis=1)


def _head_body(degp, rp, xs, wcat, cc, wout, bout2, out):
    d = degp[0] + degp[1] + 1.0
    dinv = lax.rsqrt(d)
    g = (rp[0] + rp[1] + xs[...]) * dinv
    sfull = jnp.dot(g, wcat[...], preferred_element_type=jnp.float32) + cc[...]
    z = jax.nn.sigmoid(sfull[:, :C])
    ht = jnp.tanh(sfull[:, C:])
    h = (1.0 - z) * ht
    out[...] = (jnp.dot(jnp.maximum(h, 0.0), wout[...],
                        preferred_element_type=jnp.float32) + bout2[...])


def kernel(x, edge_index, Wz, bz, Lz_w, Lz_b, Wr, br, Lr_w, Lr_b,
           Wh, bh, Lh_w, Lh_b, Wout, bout):
    f32 = jnp.float32
    src = edge_index[0].reshape(NW, K, CH)
    dst = edge_index[1].reshape(NW, K, CH)
    xpad = jnp.concatenate([x, jnp.zeros((NPAD - N, C), f32)], axis=0)
    zer = jnp.zeros((STRIPE,), f32)
    one = jnp.ones((CH,), f32)
    zrow = jnp.zeros((STRIPE, C), f32)

    degp = pl.kernel(
        _deg_body,
        out_type=jax.ShapeDtypeStruct((NC, NPAD), f32),
        mesh=_MESH,
        scratch_types=[
            pltpu.VMEM_SHARED((NPAD,), f32),
            pltpu.VMEM((K, CH), jnp.int32),
            pltpu.VMEM((CH,), f32),
        ],
    )(dst, zer, one)
    degp3 = degp.reshape(NC, NPAD, 1)

    RB = 1024
    nb = NPAD // RB
    cst = lambda bs: pl.BlockSpec(bs, lambda i: tuple(0 for _ in bs))
    xs, wcat, cc = pl.pallas_call(
        _scale_body,
        grid=(nb,),
        in_specs=[
            pl.BlockSpec((2, RB, 1), lambda i: (0, i, 0)),
            pl.BlockSpec((RB, C), lambda i: (i, 0)),
            cst((C, C)), cst((C, C)), cst((C, C)), cst((C, C)),
            cst((1, C)), cst((1, C)), cst((1, C)), cst((1, C)),
        ],
        out_specs=[
            pl.BlockSpec((RB, C), lambda i: (i, 0)),
            cst((C, 2 * C)),
            cst((1, 2 * C)),
        ],
        out_shape=[
            jax.ShapeDtypeStruct((NPAD, C), f32),
            jax.ShapeDtypeStruct((C, 2 * C), f32),
            jax.ShapeDtypeStruct((1, 2 * C), f32),
        ],
    )(degp3, xpad, Wz, Lz_w[:C], Wh, Lh_w[:C],
      bz.reshape(1, C), Lz_b.reshape(1, C), bh.reshape(1, C), Lh_b.reshape(1, C))

    rp = pl.kernel(
        _scatter_body,
        out_type=jax.ShapeDtypeStruct((NC, NPAD, C), f32),
        mesh=_MESH,
        scratch_types=[
            pltpu.VMEM_SHARED((NPAD, C), f32),
            pltpu.VMEM((K, CH), jnp.int32),
            pltpu.VMEM((K, CH), jnp.int32),
            pltpu.VMEM((CH, C), f32),
            pltpu.SemaphoreType.DMA,
        ],
    )(src, dst, xs, zrow)

    out = pl.pallas_call(
        _head_body,
        grid=(nb,),
        in_specs=[
            pl.BlockSpec((2, RB, 1), lambda i: (0, i, 0)),
            pl.BlockSpec((2, RB, C), lambda i: (0, i, 0)),
            pl.BlockSpec((RB, C), lambda i: (i, 0)),
            cst((C, 2 * C)),
            cst((1, 2 * C)),
            cst((C, OUT_DIM)),
            cst((1, OUT_DIM)),
        ],
        out_specs=pl.BlockSpec((RB, OUT_DIM), lambda i: (i, 0)),
        out_shape=jax.ShapeDtypeStruct((NPAD, OUT_DIM), f32),
    )(degp3, rp, xs, wcat, cc, Wout, bout.reshape(1, OUT_DIM))
    return out[:N]


# speedup vs baseline: 51.5251x; 1.2670x over previous
import jax
import jax.numpy as jnp
from jax import lax
from jax.experimental import pallas as pl
from jax.experimental.pallas import tpu as pltpu
from jax.experimental.pallas import tpu_sc as plsc

N = 10000
C = 128
E = 320000
OUT_DIM = 45

NC = 2
NS = 16
NW = NC * NS
NPAD = 10240
STRIPE = NPAD // NS
EPW = E // NW

CH = 80
K = EPW // CH
U = 5

CH2 = 100
K2 = EPW // CH2
U2 = 2
NG = K2 // U2

_MESH = plsc.VectorSubcoreMesh(core_axis_name="c", subcore_axis_name="s")


def _deg_body(dst_hbm, zer_hbm, one_hbm, out_hbm, deg_sp, idx_v, one_v, sem):
    c = lax.axis_index("c")
    s = lax.axis_index("s")
    w = c * NS + s
    pltpu.sync_copy(zer_hbm, deg_sp.at[pl.ds(s * STRIPE, STRIPE)])
    pltpu.sync_copy(one_hbm, one_v)
    pltpu.sync_copy(dst_hbm.at[w], idx_v)
    plsc.subcore_barrier()

    def body(j, carry):
        base = j * U
        descs = [
            pltpu.async_copy(one_v, deg_sp.at[idx_v.at[base + i]], sem,
                             add=True)
            for i in range(U)
        ]
        for dsc in descs:
            dsc.wait()
        return carry

    lax.fori_loop(0, K // U, body, 0)
    plsc.subcore_barrier()
    pltpu.sync_copy(deg_sp.at[pl.ds(s * STRIPE, STRIPE)],
                    out_hbm.at[c, pl.ds(s * STRIPE, STRIPE)])


def _scatter_body(src_hbm, dst_hbm, xs_hbm, zrow_hbm, out_hbm,
                  acc_sp, sidx, didx, rows_v, isem, gsems, ssem):
    c = lax.axis_index("c")
    s = lax.axis_index("s")
    w = c * NS + s
    pltpu.sync_copy(zrow_hbm, acc_sp.at[pl.ds(s * STRIPE, STRIPE), :])
    pltpu.sync_copy(src_hbm.at[w, pl.ds(0, U2), :], sidx.at[0])
    pltpu.sync_copy(dst_hbm.at[w, pl.ds(0, U2), :], didx.at[0])
    plsc.subcore_barrier()

    def body(j, carry):
        b = lax.rem(j, 2)
        nxt = lax.rem(j + 1, NG)
        nb = lax.rem(j + 1, 2)
        ip0 = pltpu.async_copy(src_hbm.at[w, pl.ds(nxt * U2, U2), :],
                               sidx.at[nb], isem)
        ip1 = pltpu.async_copy(dst_hbm.at[w, pl.ds(nxt * U2, U2), :],
                               didx.at[nb], isem)
        gds = [
            pltpu.async_copy(xs_hbm.at[sidx.at[b, i]], rows_v.at[i],
                             gsems.at[i])
            for i in range(U2)
        ]
        sds = []
        for i in range(U2):
            gds[i].wait()
            sds.append(
                pltpu.async_copy(rows_v.at[i], acc_sp.at[didx.at[b, i]],
                                 ssem, add=True))
        for dsc in sds:
            dsc.wait()
        ip0.wait()
        ip1.wait()
        return carry

    lax.fori_loop(0, NG, body, 0)
    plsc.subcore_barrier()
    pltpu.sync_copy(acc_sp.at[pl.ds(s * STRIPE, STRIPE), :],
                    out_hbm.at[c, pl.ds(s * STRIPE, STRIPE), :])


def _scale_body(degp, xr, wz, lz1, wh, lh1, bz2, lzb2, bh2, lhb2,
                xs, wcat, cc):
    d = degp[0] + degp[1] + 1.0
    dinv = lax.rsqrt(d)
    xs[...] = xr[...] * dinv

    @pl.when(pl.program_id(0) == 0)
    def _():
        az = jnp.dot(wz[...], lz1[...], preferred_element_type=jnp.float32)
        ah = jnp.dot(wh[...], lh1[...], preferred_element_type=jnp.float32)
        wcat[...] = jnp.concatenate([az, ah], axis=1)
        cz = jnp.dot(bz2[...], lz1[...], preferred_element_type=jnp.float32) + lzb2[...]
        chh = jnp.dot(bh2[...], lh1[...], preferred_element_type=jnp.float32) + lhb2[...]
        cc[...] = jnp.concatenate([cz, chh], axis=1)


def _head_body(degp, rp, xs, wcat, cc, wout, bout2, out):
    d = degp[0] + degp[1] + 1.0
    dinv = lax.rsqrt(d)
    g = (rp[0] + rp[1] + xs[...]) * dinv
    sfull = jnp.dot(g, wcat[...], preferred_element_type=jnp.float32) + cc[...]
    z = jax.nn.sigmoid(sfull[:, :C])
    ht = jnp.tanh(sfull[:, C:])
    h = (1.0 - z) * ht
    out[...] = (jnp.dot(jnp.maximum(h, 0.0), wout[...],
                        preferred_element_type=jnp.float32) + bout2[...])


def kernel(x, edge_index, Wz, bz, Lz_w, Lz_b, Wr, br, Lr_w, Lr_b,
           Wh, bh, Lh_w, Lh_b, Wout, bout):
    f32 = jnp.float32
    dstw = edge_index[1].reshape(NW, K, CH)
    src2 = edge_index[0].reshape(NW, K2, CH2)
    dst2 = edge_index[1].reshape(NW, K2, CH2)
    xpad = jnp.concatenate([x, jnp.zeros((NPAD - N, C), f32)], axis=0)
    zer = jnp.zeros((STRIPE,), f32)
    one = jnp.ones((CH,), f32)
    zrow = jnp.zeros((STRIPE, C), f32)

    degp = pl.kernel(
        _deg_body,
        out_type=jax.ShapeDtypeStruct((NC, NPAD), f32),
        mesh=_MESH,
        scratch_types=[
            pltpu.VMEM_SHARED((NPAD,), f32),
            pltpu.VMEM((K, CH), jnp.int32),
            pltpu.VMEM((CH,), f32),
            pltpu.SemaphoreType.DMA,
        ],
    )(dstw, zer, one)
    degp3 = degp.reshape(NC, NPAD, 1)

    RB = 1024
    nb = NPAD // RB
    cst = lambda bs: pl.BlockSpec(bs, lambda i: tuple(0 for _ in bs))
    xs, wcat, cc = pl.pallas_call(
        _scale_body,
        grid=(nb,),
        in_specs=[
            pl.BlockSpec((2, RB, 1), lambda i: (0, i, 0)),
            pl.BlockSpec((RB, C), lambda i: (i, 0)),
            cst((C, C)), cst((C, C)), cst((C, C)), cst((C, C)),
            cst((1, C)), cst((1, C)), cst((1, C)), cst((1, C)),
        ],
        out_specs=[
            pl.BlockSpec((RB, C), lambda i: (i, 0)),
            cst((C, 2 * C)),
            cst((1, 2 * C)),
        ],
        out_shape=[
            jax.ShapeDtypeStruct((NPAD, C), f32),
            jax.ShapeDtypeStruct((C, 2 * C), f32),
            jax.ShapeDtypeStruct((1, 2 * C), f32),
        ],
    )(degp3, xpad, Wz, Lz_w[:C], Wh, Lh_w[:C],
      bz.reshape(1, C), Lz_b.reshape(1, C), bh.reshape(1, C), Lh_b.reshape(1, C))

    rp = pl.kernel(
        _scatter_body,
        out_type=jax.ShapeDtypeStruct((NC, NPAD, C), f32),
        mesh=_MESH,
        scratch_types=[
            pltpu.VMEM_SHARED((NPAD, C), f32),
            pltpu.VMEM((2, U2, CH2), jnp.int32),
            pltpu.VMEM((2, U2, CH2), jnp.int32),
            pltpu.VMEM((U2, CH2, C), f32),
            pltpu.SemaphoreType.DMA,
            pltpu.SemaphoreType.DMA((U2,)),
            pltpu.SemaphoreType.DMA,
        ],
    )(src2, dst2, xs, zrow)

    out = pl.pallas_call(
        _head_body,
        grid=(nb,),
        in_specs=[
            pl.BlockSpec((2, RB, 1), lambda i: (0, i, 0)),
            pl.BlockSpec((2, RB, C), lambda i: (0, i, 0)),
            pl.BlockSpec((RB, C), lambda i: (i, 0)),
            cst((C, 2 * C)),
            cst((1, 2 * C)),
            cst((C, OUT_DIM)),
            cst((1, OUT_DIM)),
        ],
        out_specs=pl.BlockSpec((RB, OUT_DIM), lambda i: (i, 0)),
        out_shape=jax.ShapeDtypeStruct((NPAD, OUT_DIM), f32),
    )(degp3, rp, xs, wcat, cc, Wout, bout.reshape(1, OUT_DIM))
    return out[:N]


# speedup vs baseline: 52.1692x; 1.0125x over previous
import jax
import jax.numpy as jnp
from jax import lax
from jax.experimental import pallas as pl
from jax.experimental.pallas import tpu as pltpu
from jax.experimental.pallas import tpu_sc as plsc

N = 10000
C = 128
E = 320000
OUT_DIM = 45

NC = 2
NS = 16
NW = NC * NS
NPAD = 10240
STRIPE = NPAD // NS
EPW = E // NW

CH = 128
NCH = E // CH
K2 = NCH // NW
EXTRA = NCH - K2 * NW
NB = 2
NQ = 4

_MESH = plsc.VectorSubcoreMesh(core_axis_name="c", subcore_axis_name="s")


def _deg_body(dst_hbm, zer_hbm, one_hbm, out_hbm, deg_sp, didx, tdi, one_v,
              isem, ssem):
    c = lax.axis_index("c")
    s = lax.axis_index("s")
    w = c * NS + s

    def fire_idx(k):
        return pltpu.async_copy(dst_hbm.at[pl.ds((k * NW + w) * CH, CH)],
                                didx.at[k % NQ], isem.at[k % NQ])

    idesc = {k: fire_idx(k) for k in range(2)}
    pltpu.sync_copy(one_hbm, one_v)
    pltpu.sync_copy(zer_hbm, deg_sp.at[pl.ds(s * STRIPE, STRIPE)])
    plsc.subcore_barrier()

    sdesc = {}
    for k in range(K2):
        idesc.pop(k).wait()
        if k >= 2:
            sdesc.pop(k - 2).wait()
        if k + 2 < K2:
            idesc[k + 2] = fire_idx(k + 2)
        sdesc[k] = pltpu.async_copy(one_v, deg_sp.at[didx.at[k % NQ]],
                                    ssem.at[k % 2], add=True)
    for k in sorted(sdesc):
        sdesc.pop(k).wait()

    @pl.when(w < EXTRA)
    def _():
        pltpu.sync_copy(dst_hbm.at[pl.ds((K2 * NW + w) * CH, CH)], tdi.at[0])
        pltpu.sync_copy(one_v, deg_sp.at[tdi.at[0]], add=True)

    plsc.subcore_barrier()
    pltpu.sync_copy(deg_sp.at[pl.ds(s * STRIPE, STRIPE)],
                    out_hbm.at[c, pl.ds(s * STRIPE, STRIPE)])


def _scatter_body(src_hbm, dst_hbm, xs_hbm, zrow_hbm, out_hbm,
                  acc_sp, sidx, didx, tsi, tdi, rows_v, isem, gsem, ssem):
    c = lax.axis_index("c")
    s = lax.axis_index("s")
    w = c * NS + s

    def fire_idx(k):
        q = k % NQ
        off = (k * NW + w) * CH
        d0 = pltpu.async_copy(src_hbm.at[pl.ds(off, CH)], sidx.at[q],
                              isem.at[q])
        d1 = pltpu.async_copy(dst_hbm.at[pl.ds(off, CH)], didx.at[q],
                              isem.at[q])
        return (d0, d1)

    idesc = {k: fire_idx(k) for k in range(2)}
    pltpu.sync_copy(zrow_hbm, acc_sp.at[pl.ds(s * STRIPE, STRIPE), :])
    plsc.subcore_barrier()

    gdesc = {}
    sdesc = {}
    for k in range(K2):
        b = k % NB
        for d in idesc.pop(k):
            d.wait()
        if k >= NB:
            sdesc.pop(k - NB).wait()
            if k + NB < K2:
                idesc[k + NB] = fire_idx(k + NB)
        elif k + NB < K2:
            idesc[k + NB] = fire_idx(k + NB)
        gd = pltpu.async_copy(xs_hbm.at[sidx.at[k % NQ]], rows_v.at[b],
                              gsem.at[b])
        gd.wait()
        sdesc[k] = pltpu.async_copy(rows_v.at[b], acc_sp.at[didx.at[k % NQ]],
                                    ssem.at[b], add=True)
    for k in sorted(sdesc):
        sdesc.pop(k).wait()

    @pl.when(w < EXTRA)
    def _():
        off = (K2 * NW + w) * CH
        pltpu.sync_copy(src_hbm.at[pl.ds(off, CH)], tsi.at[0])
        pltpu.sync_copy(dst_hbm.at[pl.ds(off, CH)], tdi.at[0])
        pltpu.async_copy(xs_hbm.at[tsi.at[0]], rows_v.at[0],
                         gsem.at[0]).wait()
        pltpu.sync_copy(rows_v.at[0], acc_sp.at[tdi.at[0]], add=True)

    plsc.subcore_barrier()
    pltpu.sync_copy(acc_sp.at[pl.ds(s * STRIPE, STRIPE), :],
                    out_hbm.at[c, pl.ds(s * STRIPE, STRIPE), :])


def _scale_body(degp, xr, wz, lz1, wh, lh1, bz2, lzb2, bh2, lhb2,
                xs, wcat, cc):
    d = degp[0] + degp[1] + 1.0
    dinv = lax.rsqrt(d)
    xs[...] = xr[...] * dinv

    @pl.when(pl.program_id(0) == 0)
    def _():
        az = jnp.dot(wz[...], lz1[...], preferred_element_type=jnp.float32)
        ah = jnp.dot(wh[...], lh1[...], preferred_element_type=jnp.float32)
        wcat[...] = jnp.concatenate([az, ah], axis=1)
        cz = jnp.dot(bz2[...], lz1[...], preferred_element_type=jnp.float32) + lzb2[...]
        chh = jnp.dot(bh2[...], lh1[...], preferred_element_type=jnp.float32) + lhb2[...]
        cc[...] = jnp.concatenate([cz, chh], axis=1)


def _head_body(degp, rp, xs, wcat, cc, wout, bout2, out):
    d = degp[0] + degp[1] + 1.0
    dinv = lax.rsqrt(d)
    g = (rp[0] + rp[1] + xs[...]) * dinv
    sfull = jnp.dot(g, wcat[...], preferred_element_type=jnp.float32) + cc[...]
    z = jax.nn.sigmoid(sfull[:, :C])
    ht = jnp.tanh(sfull[:, C:])
    h = (1.0 - z) * ht
    out[...] = (jnp.dot(jnp.maximum(h, 0.0), wout[...],
                        preferred_element_type=jnp.float32) + bout2[...])


def kernel(x, edge_index, Wz, bz, Lz_w, Lz_b, Wr, br, Lr_w, Lr_b,
           Wh, bh, Lh_w, Lh_b, Wout, bout):
    f32 = jnp.float32
    src1d = edge_index[0]
    dst1d = edge_index[1]
    zer1 = jnp.zeros((STRIPE,), f32)
    one1 = jnp.ones((CH,), f32)
    zrow = jnp.zeros((STRIPE, C), f32)
    xpad = jnp.concatenate([x, jnp.zeros((NPAD - N, C), f32)], axis=0)

    degp = pl.kernel(
        _deg_body,
        out_type=jax.ShapeDtypeStruct((NC, NPAD), f32),
        mesh=_MESH,
        scratch_types=[
            pltpu.VMEM_SHARED((NPAD,), f32),
            pltpu.VMEM((NQ, CH), jnp.int32),
            pltpu.VMEM((1, CH), jnp.int32),
            pltpu.VMEM((CH,), f32),
            pltpu.SemaphoreType.DMA((NQ,)),
            pltpu.SemaphoreType.DMA((2,)),
        ],
    )(dst1d, zer1, one1)
    degp3 = degp.reshape(NC, NPAD, 1)

    RB = 1024
    nb = NPAD // RB
    cst = lambda bs: pl.BlockSpec(bs, lambda i: tuple(0 for _ in bs))
    xs, wcat, cc = pl.pallas_call(
        _scale_body,
        grid=(nb,),
        in_specs=[
            pl.BlockSpec((2, RB, 1), lambda i: (0, i, 0)),
            pl.BlockSpec((RB, C), lambda i: (i, 0)),
            cst((C, C)), cst((C, C)), cst((C, C)), cst((C, C)),
            cst((1, C)), cst((1, C)), cst((1, C)), cst((1, C)),
        ],
        out_specs=[
            pl.BlockSpec((RB, C), lambda i: (i, 0)),
            cst((C, 2 * C)),
            cst((1, 2 * C)),
        ],
        out_shape=[
            jax.ShapeDtypeStruct((NPAD, C), f32),
            jax.ShapeDtypeStruct((C, 2 * C), f32),
            jax.ShapeDtypeStruct((1, 2 * C), f32),
        ],
    )(degp3, xpad, Wz, Lz_w[:C], Wh, Lh_w[:C],
      bz.reshape(1, C), Lz_b.reshape(1, C), bh.reshape(1, C), Lh_b.reshape(1, C))

    rp = pl.kernel(
        _scatter_body,
        out_type=jax.ShapeDtypeStruct((NC, NPAD, C), f32),
        mesh=_MESH,
        scratch_types=[
            pltpu.VMEM_SHARED((NPAD, C), f32),
            pltpu.VMEM((NQ, CH), jnp.int32),
            pltpu.VMEM((NQ, CH), jnp.int32),
            pltpu.VMEM((1, CH), jnp.int32),
            pltpu.VMEM((1, CH), jnp.int32),
            pltpu.VMEM((NB, CH, C), f32),
            pltpu.SemaphoreType.DMA((NQ,)),
            pltpu.SemaphoreType.DMA((NB,)),
            pltpu.SemaphoreType.DMA((NB,)),
        ],
    )(src1d, dst1d, xs, zrow)

    out = pl.pallas_call(
        _head_body,
        grid=(nb,),
        in_specs=[
            pl.BlockSpec((2, RB, 1), lambda i: (0, i, 0)),
            pl.BlockSpec((2, RB, C), lambda i: (0, i, 0)),
            pl.BlockSpec((RB, C), lambda i: (i, 0)),
            cst((C, 2 * C)),
            cst((1, 2 * C)),
            cst((C, OUT_DIM)),
            cst((1, OUT_DIM)),
        ],
        out_specs=pl.BlockSpec((RB, OUT_DIM), lambda i: (i, 0)),
        out_shape=jax.ShapeDtypeStruct((NPAD, OUT_DIM), f32),
    )(degp3, rp, xs, wcat, cc, Wout, bout.reshape(1, OUT_DIM))
    return out[:N]


# speedup vs baseline: 66.1077x; 1.2672x over previous
import jax
import jax.numpy as jnp
from jax import lax
from jax.experimental import pallas as pl
from jax.experimental.pallas import tpu as pltpu
from jax.experimental.pallas import tpu_sc as plsc

N = 10000
C = 128
E = 320000
OUT_DIM = 45

NC = 2
NS = 16
NW = NC * NS
NPAD = 10240
STRIPE = NPAD // NS
EPW = E // NW

CH = 128
NCH = E // CH
K2 = NCH // NW
EXTRA = NCH - K2 * NW
NB = 2
NQ = 8
PF = 6

_MESH = plsc.VectorSubcoreMesh(core_axis_name="c", subcore_axis_name="s")


def _deg_body(ei_hbm, zer_hbm, one_hbm, out_hbm, deg_sp, ibuf, tbuf, one_v,
              isem, ssem):
    c = lax.axis_index("c")
    s = lax.axis_index("s")
    w = c * NS + s

    def fire_idx(k):
        return pltpu.async_copy(
            ei_hbm.at[pl.ds(0, 2), pl.ds((k * NW + w) * CH, CH)],
            ibuf.at[k % NQ], isem.at[k % NQ])

    idesc = {k: fire_idx(k) for k in range(min(PF, K2))}
    pltpu.sync_copy(one_hbm, one_v)
    pltpu.sync_copy(zer_hbm, deg_sp.at[pl.ds(s * STRIPE, STRIPE)])
    plsc.subcore_barrier()

    sdesc = {}
    for k in range(K2):
        idesc.pop(k).wait()
        if k >= 2:
            sdesc.pop(k - 2).wait()
        sdesc[k] = pltpu.async_copy(one_v, deg_sp.at[ibuf.at[k % NQ, 1]],
                                    ssem.at[k % 2], add=True)
        if k + PF < K2:
            idesc[k + PF] = fire_idx(k + PF)
    for k in sorted(sdesc):
        sdesc.pop(k).wait()

    @pl.when(w < EXTRA)
    def _():
        pltpu.sync_copy(
            ei_hbm.at[pl.ds(0, 2), pl.ds((K2 * NW + w) * CH, CH)], tbuf)
        pltpu.sync_copy(one_v, deg_sp.at[tbuf.at[1]], add=True)

    plsc.subcore_barrier()
    pltpu.sync_copy(deg_sp.at[pl.ds(s * STRIPE, STRIPE)],
                    out_hbm.at[c, pl.ds(s * STRIPE, STRIPE)])


def _scatter_body(ei_hbm, xs_hbm, zrow_hbm, out_hbm,
                  acc_sp, ibuf, tbuf, rows_v, isem, gsem, ssem):
    c = lax.axis_index("c")
    s = lax.axis_index("s")
    w = c * NS + s

    def fire_idx(k):
        return pltpu.async_copy(
            ei_hbm.at[pl.ds(0, 2), pl.ds((k * NW + w) * CH, CH)],
            ibuf.at[k % NQ], isem.at[k % NQ])

    def fire_gather(k):
        return pltpu.async_copy(xs_hbm.at[ibuf.at[k % NQ, 0]],
                                rows_v.at[k % NB], gsem.at[k % NB])

    def fire_scatter(k):
        return pltpu.async_copy(rows_v.at[k % NB],
                                acc_sp.at[ibuf.at[k % NQ, 1]],
                                ssem.at[k % NB], add=True)

    idesc = {k: fire_idx(k) for k in range(min(PF, K2))}
    pltpu.sync_copy(zrow_hbm, acc_sp.at[pl.ds(s * STRIPE, STRIPE), :])
    plsc.subcore_barrier()

    gdesc = {}
    sdesc = {}
    idesc.pop(0).wait()
    gdesc[0] = fire_gather(0)
    for k in range(K2):
        nxt = k + 1
        if nxt < K2:
            idesc.pop(nxt).wait()
            if nxt >= NB:
                sdesc.pop(nxt - NB).wait()
            gdesc[nxt] = fire_gather(nxt)
        gdesc.pop(k).wait()
        sdesc[k] = fire_scatter(k)
        if k + PF < K2:
            idesc[k + PF] = fire_idx(k + PF)
    for k in sorted(sdesc):
        sdesc.pop(k).wait()

    @pl.when(w < EXTRA)
    def _():
        pltpu.sync_copy(
            ei_hbm.at[pl.ds(0, 2), pl.ds((K2 * NW + w) * CH, CH)], tbuf)
        pltpu.async_copy(xs_hbm.at[tbuf.at[0]], rows_v.at[0],
                         gsem.at[0]).wait()
        pltpu.sync_copy(rows_v.at[0], acc_sp.at[tbuf.at[1]], add=True)

    plsc.subcore_barrier()
    pltpu.sync_copy(acc_sp.at[pl.ds(s * STRIPE, STRIPE), :],
                    out_hbm.at[c, pl.ds(s * STRIPE, STRIPE), :])


def _scale_body(degp, xr, wz, lz1, wh, lh1, bz2, lzb2, bh2, lhb2,
                xs, wcat, cc):
    d = degp[0] + degp[1] + 1.0
    dinv = lax.rsqrt(d)
    xs[...] = xr[...] * dinv

    @pl.when(pl.program_id(0) == 0)
    def _():
        az = jnp.dot(wz[...], lz1[...], preferred_element_type=jnp.float32)
        ah = jnp.dot(wh[...], lh1[...], preferred_element_type=jnp.float32)
        wcat[...] = jnp.concatenate([az, ah], axis=1)
        cz = jnp.dot(bz2[...], lz1[...], preferred_element_type=jnp.float32) + lzb2[...]
        chh = jnp.dot(bh2[...], lh1[...], preferred_element_type=jnp.float32) + lhb2[...]
        cc[...] = jnp.concatenate([cz, chh], axis=1)


def _head_body(degp, rp, xs, wcat, cc, wout, bout2, out):
    d = degp[0] + degp[1] + 1.0
    dinv = lax.rsqrt(d)
    g = (rp[0] + rp[1] + xs[...]) * dinv
    sfull = jnp.dot(g, wcat[...], preferred_element_type=jnp.float32) + cc[...]
    z = jax.nn.sigmoid(sfull[:, :C])
    ht = jnp.tanh(sfull[:, C:])
    h = (1.0 - z) * ht
    out[...] = (jnp.dot(jnp.maximum(h, 0.0), wout[...],
                        preferred_element_type=jnp.float32) + bout2[...])


def kernel(x, edge_index, Wz, bz, Lz_w, Lz_b, Wr, br, Lr_w, Lr_b,
           Wh, bh, Lh_w, Lh_b, Wout, bout):
    f32 = jnp.float32
    zer1 = jnp.zeros((STRIPE,), f32)
    one1 = jnp.ones((CH,), f32)
    zrow = jnp.zeros((STRIPE, C), f32)
    xpad = jnp.concatenate([x, jnp.zeros((NPAD - N, C), f32)], axis=0)

    degp = pl.kernel(
        _deg_body,
        out_type=jax.ShapeDtypeStruct((NC, NPAD), f32),
        mesh=_MESH,
        scratch_types=[
            pltpu.VMEM_SHARED((NPAD,), f32),
            pltpu.VMEM((NQ, 2, CH), jnp.int32),
            pltpu.VMEM((2, CH), jnp.int32),
            pltpu.VMEM((CH,), f32),
            pltpu.SemaphoreType.DMA((NQ,)),
            pltpu.SemaphoreType.DMA((2,)),
        ],
    )(edge_index, zer1, one1)
    degp3 = degp.reshape(NC, NPAD, 1)

    RB = 1024
    nb = NPAD // RB
    cst = lambda bs: pl.BlockSpec(bs, lambda i: tuple(0 for _ in bs))
    xs, wcat, cc = pl.pallas_call(
        _scale_body,
        grid=(nb,),
        in_specs=[
            pl.BlockSpec((2, RB, 1), lambda i: (0, i, 0)),
            pl.BlockSpec((RB, C), lambda i: (i, 0)),
            cst((C, C)), cst((C, C)), cst((C, C)), cst((C, C)),
            cst((1, C)), cst((1, C)), cst((1, C)), cst((1, C)),
        ],
        out_specs=[
            pl.BlockSpec((RB, C), lambda i: (i, 0)),
            cst((C, 2 * C)),
            cst((1, 2 * C)),
        ],
        out_shape=[
            jax.ShapeDtypeStruct((NPAD, C), f32),
            jax.ShapeDtypeStruct((C, 2 * C), f32),
            jax.ShapeDtypeStruct((1, 2 * C), f32),
        ],
    )(degp3, xpad, Wz, Lz_w[:C], Wh, Lh_w[:C],
      bz.reshape(1, C), Lz_b.reshape(1, C), bh.reshape(1, C), Lh_b.reshape(1, C))

    rp = pl.kernel(
        _scatter_body,
        out_type=jax.ShapeDtypeStruct((NC, NPAD, C), f32),
        mesh=_MESH,
        scratch_types=[
            pltpu.VMEM_SHARED((NPAD, C), f32),
            pltpu.VMEM((NQ, 2, CH), jnp.int32),
            pltpu.VMEM((2, CH), jnp.int32),
            pltpu.VMEM((NB, CH, C), f32),
            pltpu.SemaphoreType.DMA((NQ,)),
            pltpu.SemaphoreType.DMA((NB,)),
            pltpu.SemaphoreType.DMA((NB,)),
        ],
    )(edge_index, xs, zrow)

    out = pl.pallas_call(
        _head_body,
        grid=(nb,),
        in_specs=[
            pl.BlockSpec((2, RB, 1), lambda i: (0, i, 0)),
            pl.BlockSpec((2, RB, C), lambda i: (0, i, 0)),
            pl.BlockSpec((RB, C), lambda i: (i, 0)),
            cst((C, 2 * C)),
            cst((1, 2 * C)),
            cst((C, OUT_DIM)),
            cst((1, OUT_DIM)),
        ],
        out_specs=pl.BlockSpec((RB, OUT_DIM), lambda i: (i, 0)),
        out_shape=jax.ShapeDtypeStruct((NPAD, OUT_DIM), f32),
    )(degp3, rp, xs, wcat, cc, Wout, bout.reshape(1, OUT_DIM))
    return out[:N]


# speedup vs baseline: 71.2374x; 1.0776x over previous
import jax
import jax.numpy as jnp
from jax import lax
from jax.experimental import pallas as pl
from jax.experimental.pallas import tpu as pltpu
from jax.experimental.pallas import tpu_sc as plsc

N = 10000
C = 128
E = 320000
OUT_DIM = 45

NC = 2
NS = 16
NW = NC * NS
NPAD = 10240
STRIPE = NPAD // NS
EPW = E // NW

CH = 128
NCH = E // CH
K2 = NCH // NW
EXTRA = NCH - K2 * NW
NB = 2
NQ = 8
PF = 6

_MESH = plsc.VectorSubcoreMesh(core_axis_name="c", subcore_axis_name="s")


def _deg_body(ei_hbm, zer_hbm, one_hbm, out_hbm, deg_sp, ibuf, tbuf, one_v,
              isem, ssem):
    c = lax.axis_index("c")
    s = lax.axis_index("s")
    w = c * NS + s

    def fire_idx(k):
        return pltpu.async_copy(
            ei_hbm.at[pl.ds(0, 2), pl.ds((k * NW + w) * CH, CH)],
            ibuf.at[k % NQ], isem.at[k % NQ])

    idesc = {k: fire_idx(k) for k in range(min(PF, K2))}
    pltpu.sync_copy(one_hbm, one_v)
    pltpu.sync_copy(zer_hbm, deg_sp.at[pl.ds(s * STRIPE, STRIPE)])
    plsc.subcore_barrier()

    sdesc = {}
    for k in range(K2):
        idesc.pop(k).wait()
        if k >= 2:
            sdesc.pop(k - 2).wait()
        sdesc[k] = pltpu.async_copy(one_v, deg_sp.at[ibuf.at[k % NQ, 1]],
                                    ssem.at[k % 2], add=True)
        if k + PF < K2:
            idesc[k + PF] = fire_idx(k + PF)
    for k in sorted(sdesc):
        sdesc.pop(k).wait()

    @pl.when(w < EXTRA)
    def _():
        pltpu.sync_copy(
            ei_hbm.at[pl.ds(0, 2), pl.ds((K2 * NW + w) * CH, CH)], tbuf)
        pltpu.sync_copy(one_v, deg_sp.at[tbuf.at[1]], add=True)

    plsc.subcore_barrier()
    pltpu.sync_copy(deg_sp.at[pl.ds(s * STRIPE, STRIPE)],
                    out_hbm.at[c, pl.ds(s * STRIPE, STRIPE)])


def _scatter_body(ei_hbm, xs_hbm, zrow_hbm, out_hbm,
                  acc_sp, ibuf, tbuf, rows_v, isem, gsem, ssem):
    c = lax.axis_index("c")
    s = lax.axis_index("s")
    w = c * NS + s

    def fire_idx(k):
        return pltpu.async_copy(
            ei_hbm.at[pl.ds(0, 2), pl.ds((k * NW + w) * CH, CH)],
            ibuf.at[k % NQ], isem.at[k % NQ])

    def fire_gather(k):
        return pltpu.async_copy(xs_hbm.at[ibuf.at[k % NQ, 0]],
                                rows_v.at[k % NB], gsem.at[k % NB])

    def fire_scatter(k):
        return pltpu.async_copy(rows_v.at[k % NB],
                                acc_sp.at[ibuf.at[k % NQ, 1]],
                                ssem.at[k % NB], add=True)

    idesc = {k: fire_idx(k) for k in range(min(PF, K2))}
    pltpu.sync_copy(zrow_hbm, acc_sp.at[pl.ds(s * STRIPE, STRIPE), :])
    plsc.subcore_barrier()

    gdesc = {}
    sdesc = {}
    idesc.pop(0).wait()
    gdesc[0] = fire_gather(0)
    for k in range(K2):
        nxt = k + 1
        if nxt < K2:
            idesc.pop(nxt).wait()
            if nxt >= NB:
                sdesc.pop(nxt - NB).wait()
            gdesc[nxt] = fire_gather(nxt)
        gdesc.pop(k).wait()
        sdesc[k] = fire_scatter(k)
        if k + PF < K2:
            idesc[k + PF] = fire_idx(k + PF)
    for k in sorted(sdesc):
        sdesc.pop(k).wait()

    @pl.when(w < EXTRA)
    def _():
        pltpu.sync_copy(
            ei_hbm.at[pl.ds(0, 2), pl.ds((K2 * NW + w) * CH, CH)], tbuf)
        pltpu.async_copy(xs_hbm.at[tbuf.at[0]], rows_v.at[0],
                         gsem.at[0]).wait()
        pltpu.sync_copy(rows_v.at[0], acc_sp.at[tbuf.at[1]], add=True)

    plsc.subcore_barrier()
    pltpu.sync_copy(acc_sp.at[pl.ds(s * STRIPE, STRIPE), :],
                    out_hbm.at[c, pl.ds(s * STRIPE, STRIPE), :])


def _scale_body(degp, xr, wz, lz1, wh, lh1, bz2, lzb2, bh2, lhb2,
                xs, wcat, cc):
    d = degp[0] + degp[1] + 1.0
    dinv = lax.rsqrt(d).reshape(-1, 1)
    xs[...] = xr[...] * dinv

    @pl.when(pl.program_id(0) == 0)
    def _():
        az = jnp.dot(wz[...], lz1[...], preferred_element_type=jnp.float32)
        ah = jnp.dot(wh[...], lh1[...], preferred_element_type=jnp.float32)
        wcat[...] = jnp.concatenate([az, ah], axis=1)
        cz = jnp.dot(bz2[...], lz1[...], preferred_element_type=jnp.float32) + lzb2[...]
        chh = jnp.dot(bh2[...], lh1[...], preferred_element_type=jnp.float32) + lhb2[...]
        cc[...] = jnp.concatenate([cz, chh], axis=1)


def _head_body(degp, rp, xs, wcat, cc, wout, bout2, out):
    d = degp[0] + degp[1] + 1.0
    dinv = lax.rsqrt(d).reshape(-1, 1)
    g = (rp[0] + rp[1] + xs[...]) * dinv
    sfull = jnp.dot(g, wcat[...], preferred_element_type=jnp.float32) + cc[...]
    z = jax.nn.sigmoid(sfull[:, :C])
    ht = jnp.tanh(sfull[:, C:])
    h = (1.0 - z) * ht
    out[...] = (jnp.dot(jnp.maximum(h, 0.0), wout[...],
                        preferred_element_type=jnp.float32) + bout2[...])


def kernel(x, edge_index, Wz, bz, Lz_w, Lz_b, Wr, br, Lr_w, Lr_b,
           Wh, bh, Lh_w, Lh_b, Wout, bout):
    f32 = jnp.float32
    zer1 = jnp.zeros((STRIPE,), f32)
    one1 = jnp.ones((CH,), f32)
    zrow = jnp.zeros((STRIPE, C), f32)

    degp = pl.kernel(
        _deg_body,
        out_type=jax.ShapeDtypeStruct((NC, NPAD), f32),
        mesh=_MESH,
        scratch_types=[
            pltpu.VMEM_SHARED((NPAD,), f32),
            pltpu.VMEM((NQ, 2, CH), jnp.int32),
            pltpu.VMEM((2, CH), jnp.int32),
            pltpu.VMEM((CH,), f32),
            pltpu.SemaphoreType.DMA((NQ,)),
            pltpu.SemaphoreType.DMA((2,)),
        ],
    )(edge_index, zer1, one1)

    RB = 1024
    nb = NPAD // RB
    cst = lambda bs: pl.BlockSpec(bs, lambda i: tuple(0 for _ in bs))
    xs, wcat, cc = pl.pallas_call(
        _scale_body,
        grid=(nb,),
        in_specs=[
            pl.BlockSpec((2, RB), lambda i: (0, i)),
            pl.BlockSpec((RB, C), lambda i: (i, 0)),
            cst((C, C)), cst((C, C)), cst((C, C)), cst((C, C)),
            cst((1, C)), cst((1, C)), cst((1, C)), cst((1, C)),
        ],
        out_specs=[
            pl.BlockSpec((RB, C), lambda i: (i, 0)),
            cst((C, 2 * C)),
            cst((1, 2 * C)),
        ],
        out_shape=[
            jax.ShapeDtypeStruct((NPAD, C), f32),
            jax.ShapeDtypeStruct((C, 2 * C), f32),
            jax.ShapeDtypeStruct((1, 2 * C), f32),
        ],
    )(degp, x, Wz, Lz_w[:C], Wh, Lh_w[:C],
      bz.reshape(1, C), Lz_b.reshape(1, C), bh.reshape(1, C), Lh_b.reshape(1, C))

    rp = pl.kernel(
        _scatter_body,
        out_type=jax.ShapeDtypeStruct((NC, NPAD, C), f32),
        mesh=_MESH,
        scratch_types=[
            pltpu.VMEM_SHARED((NPAD, C), f32),
            pltpu.VMEM((NQ, 2, CH), jnp.int32),
            pltpu.VMEM((2, CH), jnp.int32),
            pltpu.VMEM((NB, CH, C), f32),
            pltpu.SemaphoreType.DMA((NQ,)),
            pltpu.SemaphoreType.DMA((NB,)),
            pltpu.SemaphoreType.DMA((NB,)),
        ],
    )(edge_index, xs, zrow)

    out = pl.pallas_call(
        _head_body,
        grid=(nb,),
        in_specs=[
            pl.BlockSpec((2, RB), lambda i: (0, i)),
            pl.BlockSpec((2, RB, C), lambda i: (0, i, 0)),
            pl.BlockSpec((RB, C), lambda i: (i, 0)),
            cst((C, 2 * C)),
            cst((1, 2 * C)),
            cst((C, OUT_DIM)),
            cst((1, OUT_DIM)),
        ],
        out_specs=pl.BlockSpec((RB, OUT_DIM), lambda i: (i, 0)),
        out_shape=jax.ShapeDtypeStruct((N, OUT_DIM), f32),
    )(degp, rp, xs, wcat, cc, Wout, bout.reshape(1, OUT_DIM))
    return out
